# baseline (speedup 1.0000x reference)
import jax
import jax.numpy as jnp
from jax.experimental import pallas as pl
from jax.experimental.pallas import tpu as pltpu

N = 10000
D = 64
K = N
BM = 200
NBUF = 4


def _matmul_block(a_hbm, e_ref, o_ref, abuf, sem):
    i = pl.program_id(0)
    nsteps = pl.num_programs(0)

    @pl.when(i == 0)
    def _prologue():
        for b in range(NBUF):
            pltpu.make_async_copy(
                a_hbm.at[pl.ds(b * BM, BM), :], abuf.at[b], sem.at[b]
            ).start()

    slot = jax.lax.rem(i, NBUF)
    pltpu.make_async_copy(
        a_hbm.at[pl.ds(i * BM, BM), :], abuf.at[slot], sem.at[slot]
    ).wait()

    o_ref[...] = jnp.dot(
        abuf[slot],
        e_ref[...],
        precision=jax.lax.Precision.DEFAULT,
        preferred_element_type=jnp.float32,
    )

    nxt = i + NBUF

    @pl.when(nxt < nsteps)
    def _refill():
        pltpu.make_async_copy(
            a_hbm.at[pl.ds(nxt * BM, BM), :], abuf.at[slot], sem.at[slot]
        ).start()


def kernel(A_hat, E):
    return pl.pallas_call(
        _matmul_block,
        grid=(N // BM,),
        in_specs=[
            pl.BlockSpec(memory_space=pltpu.MemorySpace.HBM),
            pl.BlockSpec((K, D), lambda i: (0, 0)),
        ],
        out_specs=pl.BlockSpec((BM, D), lambda i: (i, 0)),
        out_shape=jax.ShapeDtypeStruct((N, D), jnp.float32),
        scratch_shapes=[
            pltpu.VMEM((NBUF, BM, K), jnp.float32),
            pltpu.SemaphoreType.DMA((NBUF,)),
        ],
        compiler_params=pltpu.CompilerParams(
            dimension_semantics=("arbitrary",),
        ),
    )(A_hat, E)


# speedup vs baseline: 1.0182x; 1.0182x over previous
import jax
import jax.numpy as jnp
from jax.experimental import pallas as pl
from jax.experimental.pallas import tpu as pltpu

N = 10000
D = 64
BM = 200


def _matmul_block(a_ref, e_ref, o_ref):
    o_ref[...] = jnp.dot(
        a_ref[...], e_ref[...], preferred_element_type=jnp.float32
    )


def kernel(A_hat, E):
    return pl.pallas_call(
        _matmul_block,
        grid=(N // BM,),
        in_specs=[
            pl.BlockSpec((BM, N), lambda i: (i, 0)),
            pl.BlockSpec((N, D), lambda i: (0, 0)),
        ],
        out_specs=pl.BlockSpec((BM, D), lambda i: (i, 0)),
        out_shape=jax.ShapeDtypeStruct((N, D), jnp.float32),
        compiler_params=pltpu.CompilerParams(
            dimension_semantics=("parallel",),
        ),
    )(A_hat, E)
